# baseline (speedup 1.0000x reference)
import jax
import jax.numpy as jnp
from jax.experimental import pallas as pl
from jax.experimental.pallas import tpu as pltpu

_NUM_CLASSES = 6
_KDIM = 128
_CH = 32
_K = 4


def _body(pr_ref, wq_ref, bq_ref, x_hbm, o_hbm, q_ref,
          inbuf, outbuf, insem, outsem):
    nb = x_hbm.shape[0] // _CH

    def in_copy(t):
        return pltpu.make_async_copy(
            x_hbm.at[pl.ds(t * _CH, _CH)], inbuf.at[t % _K], insem.at[t % _K])

    def out_copy(t):
        return pltpu.make_async_copy(
            outbuf.at[t % _K], o_hbm.at[pl.ds(t * _CH, _CH)], outsem.at[t % _K])

    for t in range(_K):
        in_copy(t).start()

    for i in range(_NUM_CLASSES):
        for j in range(2):
            p = pr_ref[:, i, j, :]
            q = jax.lax.dot_general(
                p, wq_ref[i], (((1,), (1,)), ((), ())),
                preferred_element_type=jnp.float32,
            )
            q_ref[i * 2 + j] = q + bq_ref[i][None, :]

    for t in range(nb):
        s = t % _K
        in_copy(t).wait()
        if t >= _K:
            out_copy(t - _K).wait()
        outbuf[s] = inbuf[s] * jnp.float32(1.0 / 12.0)
        out_copy(t).start()
        if t + _K < nb:
            in_copy(t + _K).start()

    for t in range(nb - _K, nb):
        out_copy(t).wait()


def kernel(prototypes, assp_features, DomainTrain, Wk, bk, Wq, bq):
    b, c, h, w = assp_features.shape
    nc = prototypes.shape[1]
    pn = prototypes.shape[2]
    npairs = nc * pn

    xt = jnp.transpose(assp_features, (0, 2, 3, 1)).reshape(b * h, w, c)

    out_t, q_all = pl.pallas_call(
        _body,
        in_specs=[
            pl.BlockSpec(memory_space=pltpu.VMEM),
            pl.BlockSpec(memory_space=pltpu.VMEM),
            pl.BlockSpec(memory_space=pltpu.VMEM),
            pl.BlockSpec(memory_space=pl.ANY),
        ],
        out_specs=[
            pl.BlockSpec(memory_space=pl.ANY),
            pl.BlockSpec(memory_space=pltpu.VMEM),
        ],
        out_shape=[
            jax.ShapeDtypeStruct((b * h, w, c), jnp.float32),
            jax.ShapeDtypeStruct((npairs, b, _KDIM), jnp.float32),
        ],
        scratch_shapes=[
            pltpu.VMEM((_K, _CH, w, c), jnp.float32),
            pltpu.VMEM((_K, _CH, w, c), jnp.float32),
            pltpu.SemaphoreType.DMA((_K,)),
            pltpu.SemaphoreType.DMA((_K,)),
        ],
    )(prototypes, Wq, bq, xt)

    out = jnp.transpose(out_t.reshape(b, h, w, c), (0, 3, 1, 2))
    return (out,) + tuple(q_all[p] for p in range(npairs))
